# baseline (speedup 1.0000x reference)
import jax
import jax.numpy as jnp
from jax.experimental import pallas as pl
from jax.experimental.pallas import tpu as pltpu

_ROWS = 1000000
_COLS = 64
_BC = 16384


def _relu_scatter_body(x_ref, y_ref, c_ref, o_ref):
    o_ref[...] = jnp.maximum(c_ref[...], 0.0)

    @pl.when(pl.program_id(0) == 0)
    def _():
        m = x_ref[...] * y_ref[...]
        mt = jnp.transpose(m, (1, 0))
        o_ref[:, 0:1] = jnp.maximum(mt, 0.0)


def kernel(x, y, cache):
    ct = jnp.transpose(cache)
    grid = pl.cdiv(_ROWS, _BC)
    out_t = pl.pallas_call(
        _relu_scatter_body,
        grid=(grid,),
        in_specs=[
            pl.BlockSpec((1, _COLS), lambda i: (0, 0)),
            pl.BlockSpec((1, _COLS), lambda i: (0, 0)),
            pl.BlockSpec((_COLS, _BC), lambda i: (0, i)),
        ],
        out_specs=pl.BlockSpec((_COLS, _BC), lambda i: (0, i)),
        out_shape=jax.ShapeDtypeStruct((_COLS, _ROWS), jnp.float32),
        compiler_params=pltpu.CompilerParams(
            dimension_semantics=("arbitrary",),
        ),
    )(x, y, ct)
    return jnp.transpose(out_t)


# speedup vs baseline: 1.0241x; 1.0241x over previous
import jax
import jax.numpy as jnp
from jax.experimental import pallas as pl
from jax.experimental.pallas import tpu as pltpu

_ROWS = 1000000
_COLS = 64
_BC = 40960


def _relu_scatter_body(x_ref, y_ref, c_ref, o_ref):
    o_ref[...] = jnp.maximum(c_ref[...], 0.0)

    @pl.when(pl.program_id(0) == 0)
    def _():
        m = x_ref[...] * y_ref[...]
        mt = jnp.transpose(m, (1, 0))
        o_ref[:, 0:1] = jnp.maximum(mt, 0.0)


def kernel(x, y, cache):
    ct = jnp.transpose(cache)
    grid = pl.cdiv(_ROWS, _BC)
    out_t = pl.pallas_call(
        _relu_scatter_body,
        grid=(grid,),
        in_specs=[
            pl.BlockSpec((1, _COLS), lambda i: (0, 0)),
            pl.BlockSpec((1, _COLS), lambda i: (0, 0)),
            pl.BlockSpec((_COLS, _BC), lambda i: (0, i)),
        ],
        out_specs=pl.BlockSpec((_COLS, _BC), lambda i: (0, i)),
        out_shape=jax.ShapeDtypeStruct((_COLS, _ROWS), jnp.float32),
        compiler_params=pltpu.CompilerParams(
            dimension_semantics=("arbitrary",),
        ),
    )(x, y, ct)
    return jnp.transpose(out_t)


# speedup vs baseline: 1.0257x; 1.0015x over previous
import jax
import jax.numpy as jnp
from jax.experimental import pallas as pl
from jax.experimental.pallas import tpu as pltpu

_ROWS = 1000000
_COLS = 64
_BC = 51200


def _relu_scatter_body(x_ref, y_ref, c_ref, o_ref):
    o_ref[...] = jnp.maximum(c_ref[...], 0.0)

    @pl.when(pl.program_id(0) == 0)
    def _():
        m = x_ref[...] * y_ref[...]
        mt = jnp.transpose(m, (1, 0))
        o_ref[:, 0:1] = jnp.maximum(mt, 0.0)


def kernel(x, y, cache):
    ct = jnp.transpose(cache)
    grid = pl.cdiv(_ROWS, _BC)
    out_t = pl.pallas_call(
        _relu_scatter_body,
        grid=(grid,),
        in_specs=[
            pl.BlockSpec((1, _COLS), lambda i: (0, 0)),
            pl.BlockSpec((1, _COLS), lambda i: (0, 0)),
            pl.BlockSpec((_COLS, _BC), lambda i: (0, i)),
        ],
        out_specs=pl.BlockSpec((_COLS, _BC), lambda i: (0, i)),
        out_shape=jax.ShapeDtypeStruct((_COLS, _ROWS), jnp.float32),
        compiler_params=pltpu.CompilerParams(
            dimension_semantics=("arbitrary",),
        ),
    )(x, y, ct)
    return jnp.transpose(out_t)


# speedup vs baseline: 1.0264x; 1.0007x over previous
import jax
import jax.numpy as jnp
from jax.experimental import pallas as pl
from jax.experimental.pallas import tpu as pltpu

_ROWS = 1000000
_COLS = 64
_BC = 57344


def _relu_scatter_body(x_ref, y_ref, c_ref, o_ref):
    o_ref[...] = jnp.maximum(c_ref[...], 0.0)

    @pl.when(pl.program_id(0) == 0)
    def _():
        m = x_ref[...] * y_ref[...]
        mt = jnp.transpose(m, (1, 0))
        o_ref[:, 0:1] = jnp.maximum(mt, 0.0)


def kernel(x, y, cache):
    ct = jnp.transpose(cache)
    grid = pl.cdiv(_ROWS, _BC)
    out_t = pl.pallas_call(
        _relu_scatter_body,
        grid=(grid,),
        in_specs=[
            pl.BlockSpec((1, _COLS), lambda i: (0, 0)),
            pl.BlockSpec((1, _COLS), lambda i: (0, 0)),
            pl.BlockSpec((_COLS, _BC), lambda i: (0, i)),
        ],
        out_specs=pl.BlockSpec((_COLS, _BC), lambda i: (0, i)),
        out_shape=jax.ShapeDtypeStruct((_COLS, _ROWS), jnp.float32),
        compiler_params=pltpu.CompilerParams(
            dimension_semantics=("arbitrary",),
            vmem_limit_bytes=67108864,
        ),
    )(x, y, ct)
    return jnp.transpose(out_t)


# speedup vs baseline: 1.0264x; 1.0000x over previous
import jax
import jax.numpy as jnp
from jax.experimental import pallas as pl
from jax.experimental.pallas import tpu as pltpu

_ROWS = 1000000
_COLS = 64
_BC = 61440


def _relu_scatter_body(x_ref, y_ref, c_ref, o_ref):
    o_ref[...] = jnp.maximum(c_ref[...], 0.0)

    @pl.when(pl.program_id(0) == 0)
    def _():
        m = x_ref[...] * y_ref[...]
        mt = jnp.transpose(m, (1, 0))
        o_ref[:, 0:1] = jnp.maximum(mt, 0.0)


def kernel(x, y, cache):
    ct = jnp.transpose(cache)
    grid = pl.cdiv(_ROWS, _BC)
    out_t = pl.pallas_call(
        _relu_scatter_body,
        grid=(grid,),
        in_specs=[
            pl.BlockSpec((1, _COLS), lambda i: (0, 0)),
            pl.BlockSpec((1, _COLS), lambda i: (0, 0)),
            pl.BlockSpec((_COLS, _BC), lambda i: (0, i)),
        ],
        out_specs=pl.BlockSpec((_COLS, _BC), lambda i: (0, i)),
        out_shape=jax.ShapeDtypeStruct((_COLS, _ROWS), jnp.float32),
        compiler_params=pltpu.CompilerParams(
            dimension_semantics=("arbitrary",),
            vmem_limit_bytes=67108864,
        ),
    )(x, y, ct)
    return jnp.transpose(out_t)
